# baseline (speedup 1.0000x reference)
import jax
import jax.numpy as jnp
from jax.experimental import pallas as pl
from jax.experimental.pallas import tpu as pltpu

N = 4096
D_IN = 512
D_OUT = 512
BI = 512
NI = N // BI
HK = N // 2


def _fused_kernel(x_ref, w_ref, adj1_ref, adj2_ref, o_ref, h_ref):
    s = pl.program_id(0)

    @pl.when(s == 0)
    def _build_h():
        h_ref[...] = jnp.dot(
            x_ref[...], w_ref[...], preferred_element_type=jnp.float32
        ).astype(jnp.bfloat16)

    @pl.when(s > 0)
    def _aggregate():
        a1 = adj1_ref[...]
        a2 = adj2_ref[...]
        deg = jnp.sum(a1, axis=1, keepdims=True) + jnp.sum(
            a2, axis=1, keepdims=True
        )
        acc = jnp.dot(
            a1.astype(jnp.bfloat16), h_ref[:HK, :],
            preferred_element_type=jnp.float32,
        ) + jnp.dot(
            a2.astype(jnp.bfloat16), h_ref[HK:, :],
            preferred_element_type=jnp.float32,
        )
        o_ref[...] = acc / deg


@jax.jit
def kernel(input, adj, W):
    return pl.pallas_call(
        _fused_kernel,
        grid=(NI + 1,),
        in_specs=[
            pl.BlockSpec((N, D_IN), lambda s: (0, 0)),
            pl.BlockSpec((D_IN, D_OUT), lambda s: (0, 0)),
            pl.BlockSpec((BI, HK), lambda s: (jnp.maximum(s - 1, 0), 0)),
            pl.BlockSpec((BI, HK), lambda s: (jnp.maximum(s - 1, 0), 1)),
        ],
        out_specs=pl.BlockSpec(
            (BI, D_OUT), lambda s: (jnp.maximum(s - 1, 0), 0)
        ),
        out_shape=jax.ShapeDtypeStruct((N, D_OUT), jnp.float32),
        scratch_shapes=[
            pltpu.VMEM((N, D_OUT), jnp.bfloat16),
        ],
        compiler_params=pltpu.CompilerParams(
            dimension_semantics=("arbitrary",),
        ),
    )(input, W, adj, adj)


# speedup vs baseline: 1.0613x; 1.0613x over previous
import jax
import jax.numpy as jnp
from jax.experimental import pallas as pl
from jax.experimental.pallas import tpu as pltpu

N = 4096
D_IN = 512
D_OUT = 512
BI = 512
NI = N // BI


def _fused_kernel(x_hbm, w_hbm, adj_hbm, o_hbm,
                  x_v, w_v, h_v, a0, a1, o0, o1,
                  sx, sw, sa0, sa1, so0, so1):
    abuf = [a0, a1]
    asem = [sa0, sa1]
    obuf = [o0, o1]
    osem = [so0, so1]

    cx = pltpu.make_async_copy(x_hbm, x_v, sx)
    cx.start()
    cw = pltpu.make_async_copy(w_hbm, w_v, sw)
    cw.start()
    cx.wait()
    cw.wait()

    for j in range(min(2, NI)):
        pltpu.make_async_copy(
            adj_hbm.at[pl.ds(j * BI, BI), :], abuf[j], asem[j]
        ).start()

    h_v[...] = jnp.dot(
        x_v[...], w_v[...], preferred_element_type=jnp.float32
    ).astype(jnp.bfloat16)

    for i in range(NI):
        b = i % 2
        pltpu.make_async_copy(
            adj_hbm.at[pl.ds(i * BI, BI), :], abuf[b], asem[b]
        ).wait()
        a = abuf[b][...]
        deg = jnp.sum(a, axis=1, keepdims=True)
        acc = jnp.dot(
            a.astype(jnp.bfloat16), h_v[...],
            preferred_element_type=jnp.float32,
        )
        if i >= 2:
            pltpu.make_async_copy(
                obuf[b], o_hbm.at[pl.ds((i - 2) * BI, BI), :], osem[b]
            ).wait()
        obuf[b][...] = acc / deg
        pltpu.make_async_copy(
            obuf[b], o_hbm.at[pl.ds(i * BI, BI), :], osem[b]
        ).start()
        if i + 2 < NI:
            pltpu.make_async_copy(
                adj_hbm.at[pl.ds((i + 2) * BI, BI), :], abuf[b], asem[b]
            ).start()

    for i in (NI - 2, NI - 1):
        b = i % 2
        pltpu.make_async_copy(
            obuf[b], o_hbm.at[pl.ds(i * BI, BI), :], osem[b]
        ).wait()


@jax.jit
def kernel(input, adj, W):
    return pl.pallas_call(
        _fused_kernel,
        in_specs=[
            pl.BlockSpec(memory_space=pltpu.MemorySpace.HBM),
            pl.BlockSpec(memory_space=pltpu.MemorySpace.HBM),
            pl.BlockSpec(memory_space=pltpu.MemorySpace.HBM),
        ],
        out_specs=pl.BlockSpec(memory_space=pltpu.MemorySpace.HBM),
        out_shape=jax.ShapeDtypeStruct((N, D_OUT), jnp.float32),
        scratch_shapes=[
            pltpu.VMEM((N, D_IN), jnp.float32),
            pltpu.VMEM((D_IN, D_OUT), jnp.float32),
            pltpu.VMEM((N, D_OUT), jnp.bfloat16),
            pltpu.VMEM((BI, N), jnp.float32),
            pltpu.VMEM((BI, N), jnp.float32),
            pltpu.VMEM((BI, D_OUT), jnp.float32),
            pltpu.VMEM((BI, D_OUT), jnp.float32),
            pltpu.SemaphoreType.DMA,
            pltpu.SemaphoreType.DMA,
            pltpu.SemaphoreType.DMA,
            pltpu.SemaphoreType.DMA,
            pltpu.SemaphoreType.DMA,
            pltpu.SemaphoreType.DMA,
        ],
    )(input, W, adj)
